# Initial kernel scaffold; baseline (speedup 1.0000x reference)
import jax
import jax.numpy as jnp
from jax.experimental import pallas as pl


def kernel(input_ids, embed_table):
    raise NotImplementedError("write your pallas kernel here")


# speedup vs baseline: 1.7648x; 1.7648x over previous
import functools

import jax
import jax.numpy as jnp
from jax import lax
from jax.experimental import pallas as pl
from jax.experimental.pallas import tpu as pltpu
from jax.experimental.pallas import tpu_sc as plsc

D_MODEL = 4096
NUM_CORES = 2
NUM_SUBCORES = 16
NUM_WORKERS = NUM_CORES * NUM_SUBCORES
CHUNK = 8
NBUF = 2


def _emb_body(n_chunks, b_per_w, idx_hbm, table_hbm, out_hbm, idx_v, rows_v, gsem):
    wid = lax.axis_index("s") * NUM_CORES + lax.axis_index("c")
    base = wid * b_per_w

    pltpu.sync_copy(idx_hbm.at[wid], idx_v)

    for b in range(NBUF):
        pltpu.async_copy(table_hbm.at[idx_v.at[b]], rows_v.at[b], gsem.at[b])

    @pl.loop(0, n_chunks - NBUF, step=NBUF)
    def _main(jv):
        for b in range(NBUF):
            j = jv + b
            pltpu.make_async_copy(
                table_hbm.at[idx_v.at[0]], rows_v.at[b], gsem.at[b]
            ).wait()
            pltpu.sync_copy(rows_v.at[b], out_hbm.at[pl.ds(base + j * CHUNK, CHUNK)])
            pltpu.async_copy(
                table_hbm.at[idx_v.at[j + NBUF]], rows_v.at[b], gsem.at[b]
            )

    for b in range(NBUF):
        j = n_chunks - NBUF + b
        pltpu.make_async_copy(
            table_hbm.at[idx_v.at[0]], rows_v.at[b], gsem.at[b]
        ).wait()
        pltpu.sync_copy(rows_v.at[b], out_hbm.at[pl.ds(base + j * CHUNK, CHUNK)])


@functools.partial(jax.jit, static_argnames=("n_tokens",))
def _embed(idx_grouped, embed_table, n_tokens):
    b_per_w = n_tokens // NUM_WORKERS
    n_chunks = b_per_w // CHUNK
    mesh = plsc.VectorSubcoreMesh(
        core_axis_name="c",
        subcore_axis_name="s",
        num_cores=NUM_CORES,
        num_subcores=NUM_SUBCORES,
    )
    run = pl.kernel(
        functools.partial(_emb_body, n_chunks, b_per_w),
        out_type=jax.ShapeDtypeStruct((n_tokens, D_MODEL), jnp.float32),
        mesh=mesh,
        scratch_types=[
            pltpu.VMEM((n_chunks, CHUNK), jnp.int32),
            pltpu.VMEM((NBUF, CHUNK, D_MODEL), jnp.float32),
            pltpu.SemaphoreType.DMA((NBUF,)),
        ],
    )
    return run(idx_grouped, embed_table)


def kernel(input_ids, embed_table):
    batch, seq = input_ids.shape
    n_tokens = batch * seq
    b_per_w = n_tokens // NUM_WORKERS
    n_chunks = b_per_w // CHUNK
    idx_grouped = input_ids.reshape(NUM_WORKERS, n_chunks, CHUNK).astype(jnp.int32)
    out = _embed(idx_grouped, embed_table, n_tokens)
    return out.reshape(batch, seq, embed_table.shape[1])
